# Initial kernel scaffold; baseline (speedup 1.0000x reference)
import jax
import jax.numpy as jnp
from jax.experimental import pallas as pl


def kernel(x, edge_index, edge_attr, batch, W_embed, b_embed, tp_w, mlp_w1, mlp_b1, mlp_w2, mlp_b2, lin_w, lin_b):
    raise NotImplementedError("write your pallas kernel here")


# speedup vs baseline: 2.1331x; 2.1331x over previous
import functools

import jax
import jax.numpy as jnp
from jax import lax
from jax.experimental import pallas as pl
from jax.experimental.pallas import tpu as pltpu
from jax.experimental.pallas import tpu_sc as plsc

N = 50000
E = 800000
G = 64
MUL = 16
DIM = 48
LAYERS = 4

NC = 2
NS = 16
NW = NC * NS

CHUNK = 128
SUPER = 10
SS = SUPER * CHUNK
NSS = E // SS
SUPER_W = 5
SS_W = SUPER_W * CHUNK
NSS_W = E // SS_W
HALF = N // NC
HALF_PAD = HALF + 8
ROWCH = 250
NROWCH = HALF // ROWCH

BE = 1600
BN = 1000

E0 = 409600
E1 = E - E0


def _dot(a, b):
    return jnp.dot(a, b, preferred_element_type=jnp.float32,
                   precision=lax.Precision.HIGHEST)


def _embed_body(x_ref, w_ref, b_ref, o_ref):
    o_ref[...] = _dot(x_ref[...], w_ref[...]) + b_ref[...]


def _edgeprep_body(ea_ref, es_ref, sh_ref):
    ea = ea_ref[...]
    r = ea[:, 2:5]
    r2 = jnp.sum(r * r, axis=1, keepdims=True)
    z = jnp.zeros((BE, 1), jnp.float32)
    es_ref[...] = jnp.concatenate([ea[:, 0:2], r2, z], axis=1)
    inv = jnp.sqrt(3.0) / (jnp.sqrt(r2) + 1e-12)
    sh_ref[...] = jnp.concatenate([r * inv, z], axis=1)


def _dot_h(a, b):
    return jnp.dot(a, b, preferred_element_type=jnp.float32)


def _msg_body(xj_ref, es_ref, sh_ref, w1_ref, b1_ref, w2_ref, b2_ref,
              Wtp_ref, o_ref):
    es = es_ref[...]
    t1 = _dot_h(es, w1_ref[...]) + b1_ref[...]
    g1 = t1 * jax.nn.sigmoid(t1)
    gate = _dot_h(g1, w2_ref[...]) + b2_ref[...]
    xj = xj_ref[...]
    sh = sh_ref[...]
    a0, a1, a2 = xj[:, 0:16], xj[:, 16:32], xj[:, 32:48]
    b0, b1, b2 = sh[:, 0:1], sh[:, 1:2], sh[:, 2:3]
    cg = jnp.concatenate([a1 * b2 - a2 * b1,
                          a2 * b0 - a0 * b2,
                          a0 * b1 - a1 * b0], axis=1)
    tp = _dot_h(cg, Wtp_ref[...])
    o_ref[...] = gate * tp


def _pool_body(h_ref, b_ref, lw_ref, lb_ref, o_ref, acc_ref):
    i = pl.program_id(0)

    @pl.when(i == 0)
    def _():
        acc_ref[...] = jnp.zeros_like(acc_ref)

    bb = b_ref[0]
    gi = lax.broadcasted_iota(jnp.int32, (G, BN), 0)
    oh = (gi == bb).astype(jnp.float32)
    h_aug = jnp.concatenate(
        [h_ref[...], jnp.ones((BN, 1), jnp.float32)], axis=1)
    acc_ref[...] += _dot(oh, h_aug)

    @pl.when(i == pl.num_programs(0) - 1)
    def _():
        acc = acc_ref[...]
        pooled = acc[:, :DIM] / jnp.maximum(acc[:, DIM:DIM + 1], 1.0)
        o_ref[...] = _dot(pooled, lw_ref[...]) + lb_ref[...]


def _embed(x, W, b):
    return pl.pallas_call(
        _embed_body,
        grid=(N // BN,),
        in_specs=[pl.BlockSpec((BN, 3), lambda i: (i, 0)),
                  pl.BlockSpec((3, DIM), lambda i: (0, 0)),
                  pl.BlockSpec((1, DIM), lambda i: (0, 0))],
        out_specs=pl.BlockSpec((BN, DIM), lambda i: (i, 0)),
        out_shape=jax.ShapeDtypeStruct((N, DIM), jnp.float32),
        compiler_params=pltpu.CompilerParams(dimension_semantics=("parallel",)),
    )(x, W, b)


def _edgeprep(ea):
    return pl.pallas_call(
        _edgeprep_body,
        grid=(E // BE,),
        in_specs=[pl.BlockSpec((BE, 5), lambda i: (i, 0))],
        out_specs=[pl.BlockSpec((BE, 4), lambda i: (i, 0)),
                   pl.BlockSpec((BE, 4), lambda i: (i, 0))],
        out_shape=[jax.ShapeDtypeStruct((E, 4), jnp.float32),
                   jax.ShapeDtypeStruct((E, 4), jnp.float32)],
        compiler_params=pltpu.CompilerParams(dimension_semantics=("parallel",)),
    )(ea)


def _msg(xj, es, sh, w1, b1, w2, b2, Wtp):
    small = lambda shp: pl.BlockSpec(shp, lambda i: (0, 0))
    e_part = xj.shape[0]
    return pl.pallas_call(
        _msg_body,
        grid=(e_part // BE,),
        in_specs=[pl.BlockSpec((BE, DIM), lambda i: (i, 0)),
                  pl.BlockSpec((BE, 4), lambda i: (i, 0)),
                  pl.BlockSpec((BE, 4), lambda i: (i, 0)),
                  small((4, 16)), small((1, 16)), small((16, DIM)),
                  small((1, DIM)), small((DIM, DIM))],
        out_specs=pl.BlockSpec((BE, DIM), lambda i: (i, 0)),
        out_shape=jax.ShapeDtypeStruct((e_part, DIM), jnp.float32),
        compiler_params=pltpu.CompilerParams(dimension_semantics=("parallel",)),
    )(xj, es, sh, w1, b1, w2, b2, Wtp)


def _pool(h, batch3, lw, lb):
    return pl.pallas_call(
        _pool_body,
        grid=(N // BN,),
        in_specs=[pl.BlockSpec((BN, DIM), lambda i: (i, 0)),
                  pl.BlockSpec((1, 1, BN), lambda i: (i, 0, 0)),
                  pl.BlockSpec((DIM, 1), lambda i: (0, 0)),
                  pl.BlockSpec((1, 1), lambda i: (0, 0))],
        out_specs=pl.BlockSpec((G, 1), lambda i: (0, 0)),
        out_shape=jax.ShapeDtypeStruct((G, 1), jnp.float32),
        scratch_shapes=[pltpu.VMEM((G, DIM + 1), jnp.float32)],
    )(h, batch3, lw, lb)


_MESH = plsc.VectorSubcoreMesh(core_axis_name="c", subcore_axis_name="s")


def _make_gather(e_part):
    nss_tot = e_part // SS

    @functools.partial(
        pl.kernel,
        out_type=jax.ShapeDtypeStruct((e_part, DIM), jnp.float32),
        mesh=_MESH,
        scratch_types=[pltpu.VMEM((SS,), jnp.int32),
                       pltpu.VMEM((SS, DIM), jnp.float32),
                       pltpu.SemaphoreType.DMA],
        compiler_params=pltpu.CompilerParams(use_tc_tiling_on_sc=False),
    )
    def gather_sc(h_hbm, src_hbm, out_hbm, idx_v, rows_v, sem):
        c = lax.axis_index("c")
        s = lax.axis_index("s")
        wid = s * NC + c
        nss = (nss_tot - wid + NW - 1) // NW

        def body(j, carry):
            base = (wid + j * NW) * SS
            pltpu.sync_copy(src_hbm.at[pl.ds(base, SS)], idx_v)
            cps = [pltpu.async_copy(h_hbm.at[idx_v.at[pl.ds(r * CHUNK, CHUNK)]],
                                    rows_v.at[pl.ds(r * CHUNK, CHUNK)], sem)
                   for r in range(SUPER)]
            for cp in cps:
                cp.wait()
            pltpu.sync_copy(rows_v, out_hbm.at[pl.ds(base, SS)])
            return carry

        lax.fori_loop(0, nss, body, 0)

    return gather_sc


def _make_scatter(e_part):
    nss_w_tot = e_part // SS_W

    @functools.partial(
        pl.kernel,
        out_type=jax.ShapeDtypeStruct((N, DIM), jnp.float32),
        mesh=_MESH,
        scratch_types=[pltpu.VMEM((SS_W,), jnp.int32),
                       pltpu.VMEM((SUPER_W, CHUNK), jnp.int32),
                       pltpu.VMEM((SS_W, DIM), jnp.float32),
                       pltpu.VMEM((ROWCH, DIM), jnp.float32),
                       pltpu.VMEM_SHARED((HALF_PAD, DIM), jnp.float32),
                       pltpu.SemaphoreType.DMA,
                       pltpu.SemaphoreType.DMA],
        compiler_params=pltpu.CompilerParams(use_tc_tiling_on_sc=False),
    )
    def scatter_sc(h_hbm, msg_hbm, dst_hbm, out_hbm, dst_v, idx_v, msg_v,
                   stage_v, acc_sh, sem, sem2):
        c = lax.axis_index("c")
        s = lax.axis_index("s")
        node_base = c * HALF

        nrow = (NROWCH - s + NS - 1) // NS

        def seed(j, carry):
            r = (s + j * NS) * ROWCH
            pltpu.sync_copy(h_hbm.at[pl.ds(node_base + r, ROWCH)], stage_v)
            pltpu.sync_copy(stage_v, acc_sh.at[pl.ds(r, ROWCH)])
            return carry

        lax.fori_loop(0, nrow, seed, 0)
        plsc.subcore_barrier()

        nss = (nss_w_tot - s + NS - 1) // NS

        def body(j, carry):
            base = (s + j * NS) * SS_W
            cd = pltpu.async_copy(dst_hbm.at[pl.ds(base, SS_W)], dst_v, sem)
            cm = pltpu.async_copy(msg_hbm.at[pl.ds(base, SS_W)], msg_v, sem)
            cd.wait()
            for r in range(SUPER_W):
                for q in range(CHUNK // 16):
                    v = dst_v[pl.ds(r * CHUNK + q * 16, 16)] - node_base
                    ok = (v >= 0) & (v < HALF)
                    idx_v[r, pl.ds(q * 16, 16)] = jnp.where(ok, v, HALF)
            cm.wait()
            cps = [pltpu.async_copy(msg_v.at[pl.ds(r * CHUNK, CHUNK)],
                                    acc_sh.at[idx_v.at[r]], sem2, add=True)
                   for r in range(SUPER_W)]
            for cp in cps:
                cp.wait()
            return carry

        lax.fori_loop(0, nss, body, 0)
        plsc.subcore_barrier()

        def drain(j, carry):
            r = (s + j * NS) * ROWCH
            pltpu.sync_copy(acc_sh.at[pl.ds(r, ROWCH)], stage_v)
            pltpu.sync_copy(stage_v, out_hbm.at[pl.ds(node_base + r, ROWCH)])
            return carry

        lax.fori_loop(0, nrow, drain, 0)

    return scatter_sc


_GATHER_SC = (_make_gather(E0), _make_gather(E1))
_SCATTER_SC = (_make_scatter(E0), _make_scatter(E1))


@jax.jit
def kernel(x, edge_index, edge_attr, batch, W_embed, b_embed, tp_w,
           mlp_w1, mlp_b1, mlp_w2, mlp_b2, lin_w, lin_b):
    f32 = jnp.float32
    t = jnp.arange(DIM)
    orig_of_T = (t % MUL) * 3 + t // MUL
    k = t // MUL
    u = t % MUL

    W_embT = W_embed[:, orig_of_T]
    b_embT = b_embed[orig_of_T].reshape(1, DIM)
    w1p = jnp.concatenate([mlp_w1, jnp.zeros((LAYERS, 1, 16), f32)], axis=1)
    b1r = mlp_b1.reshape(LAYERS, 1, 16)
    w2T = mlp_w2[:, :, orig_of_T]
    b2T = mlp_b2[:, orig_of_T].reshape(LAYERS, 1, DIM)
    tp_norm = 1.0 / jnp.sqrt(2.0 * MUL)
    I3 = jnp.eye(3, dtype=f32)
    Wtp = jnp.einsum('ij,luv->liujv', I3, tp_w).reshape(LAYERS, DIM, DIM) * tp_norm
    lwT = lin_w[orig_of_T]
    lbr = lin_b.reshape(1, 1)

    src = edge_index[0]
    dst = edge_index[1]
    batch3 = batch.reshape(N // BN, 1, BN)

    src_h = (src[:E0], src[E0:])
    dst_h = (dst[:E0], dst[E0:])

    h = _embed(x, W_embT, b_embT)
    es, sh = _edgeprep(edge_attr)
    es_h = (es[:E0], es[E0:])
    sh_h = (sh[:E0], sh[E0:])
    for l in range(LAYERS):
        xj0 = _GATHER_SC[0](h, src_h[0])
        xj1 = _GATHER_SC[1](h, src_h[1])
        m0 = _msg(xj0, es_h[0], sh_h[0], w1p[l], b1r[l], w2T[l], b2T[l], Wtp[l])
        m1 = _msg(xj1, es_h[1], sh_h[1], w1p[l], b1r[l], w2T[l], b2T[l], Wtp[l])
        h = _SCATTER_SC[0](h, m0, dst_h[0])
        h = _SCATTER_SC[1](h, m1, dst_h[1])
    out = _pool(h, batch3, lwT, lbr)
    return out.reshape(-1)


# speedup vs baseline: 2.1439x; 1.0050x over previous
import functools

import jax
import jax.numpy as jnp
from jax import lax
from jax.experimental import pallas as pl
from jax.experimental.pallas import tpu as pltpu
from jax.experimental.pallas import tpu_sc as plsc

N = 50000
E = 800000
G = 64
MUL = 16
DIM = 48
LAYERS = 4

NC = 2
NS = 16
NW = NC * NS

CHUNK = 128
SUPER = 10
SS = SUPER * CHUNK
NSS = E // SS
SUPER_W = 5
SS_W = SUPER_W * CHUNK
NSS_W = E // SS_W
HALF = N // NC
HALF_PAD = HALF + 8
ROWCH = 250
NROWCH = HALF // ROWCH

BE = 1600
BN = 1000

E0 = 409600
E1 = E - E0
EQ = (204800, 204800, 204800, 185600)
EQ_OFF = (0, 204800, 409600, 614400)


def _dot(a, b):
    return jnp.dot(a, b, preferred_element_type=jnp.float32,
                   precision=lax.Precision.HIGHEST)


def _embed_body(x_ref, w_ref, b_ref, o_ref):
    o_ref[...] = _dot(x_ref[...], w_ref[...]) + b_ref[...]


def _edgeprep_body(ea_ref, es_ref, sh_ref):
    ea = ea_ref[...]
    r = ea[:, 2:5]
    r2 = jnp.sum(r * r, axis=1, keepdims=True)
    z = jnp.zeros((BE, 1), jnp.float32)
    es_ref[...] = jnp.concatenate([ea[:, 0:2], r2, z], axis=1)
    inv = jnp.sqrt(3.0) / (jnp.sqrt(r2) + 1e-12)
    sh_ref[...] = jnp.concatenate([r * inv, z], axis=1)


def _dot_h(a, b):
    return jnp.dot(a, b, preferred_element_type=jnp.float32)


def _msg_body(xj_ref, es_ref, sh_ref, w1_ref, b1_ref, w2_ref, b2_ref,
              Wtp_ref, o_ref):
    es = es_ref[...]
    t1 = _dot_h(es, w1_ref[...]) + b1_ref[...]
    g1 = t1 * jax.nn.sigmoid(t1)
    gate = _dot_h(g1, w2_ref[...]) + b2_ref[...]
    xj = xj_ref[...]
    sh = sh_ref[...]
    a0, a1, a2 = xj[:, 0:16], xj[:, 16:32], xj[:, 32:48]
    b0, b1, b2 = sh[:, 0:1], sh[:, 1:2], sh[:, 2:3]
    cg = jnp.concatenate([a1 * b2 - a2 * b1,
                          a2 * b0 - a0 * b2,
                          a0 * b1 - a1 * b0], axis=1)
    tp = _dot_h(cg, Wtp_ref[...])
    o_ref[...] = gate * tp


def _pool_body(h_ref, b_ref, lw_ref, lb_ref, o_ref, acc_ref):
    i = pl.program_id(0)

    @pl.when(i == 0)
    def _():
        acc_ref[...] = jnp.zeros_like(acc_ref)

    bb = b_ref[0]
    gi = lax.broadcasted_iota(jnp.int32, (G, BN), 0)
    oh = (gi == bb).astype(jnp.float32)
    h_aug = jnp.concatenate(
        [h_ref[...], jnp.ones((BN, 1), jnp.float32)], axis=1)
    acc_ref[...] += _dot(oh, h_aug)

    @pl.when(i == pl.num_programs(0) - 1)
    def _():
        acc = acc_ref[...]
        pooled = acc[:, :DIM] / jnp.maximum(acc[:, DIM:DIM + 1], 1.0)
        o_ref[...] = _dot(pooled, lw_ref[...]) + lb_ref[...]


def _embed(x, W, b):
    return pl.pallas_call(
        _embed_body,
        grid=(N // BN,),
        in_specs=[pl.BlockSpec((BN, 3), lambda i: (i, 0)),
                  pl.BlockSpec((3, DIM), lambda i: (0, 0)),
                  pl.BlockSpec((1, DIM), lambda i: (0, 0))],
        out_specs=pl.BlockSpec((BN, DIM), lambda i: (i, 0)),
        out_shape=jax.ShapeDtypeStruct((N, DIM), jnp.float32),
        compiler_params=pltpu.CompilerParams(dimension_semantics=("parallel",)),
    )(x, W, b)


def _edgeprep(ea):
    return pl.pallas_call(
        _edgeprep_body,
        grid=(E // BE,),
        in_specs=[pl.BlockSpec((BE, 5), lambda i: (i, 0))],
        out_specs=[pl.BlockSpec((BE, 4), lambda i: (i, 0)),
                   pl.BlockSpec((BE, 4), lambda i: (i, 0))],
        out_shape=[jax.ShapeDtypeStruct((E, 4), jnp.float32),
                   jax.ShapeDtypeStruct((E, 4), jnp.float32)],
        compiler_params=pltpu.CompilerParams(dimension_semantics=("parallel",)),
    )(ea)


def _msg(xj, es, sh, w1, b1, w2, b2, Wtp):
    small = lambda shp: pl.BlockSpec(shp, lambda i: (0, 0))
    e_part = xj.shape[0]
    return pl.pallas_call(
        _msg_body,
        grid=(e_part // BE,),
        in_specs=[pl.BlockSpec((BE, DIM), lambda i: (i, 0)),
                  pl.BlockSpec((BE, 4), lambda i: (i, 0)),
                  pl.BlockSpec((BE, 4), lambda i: (i, 0)),
                  small((4, 16)), small((1, 16)), small((16, DIM)),
                  small((1, DIM)), small((DIM, DIM))],
        out_specs=pl.BlockSpec((BE, DIM), lambda i: (i, 0)),
        out_shape=jax.ShapeDtypeStruct((e_part, DIM), jnp.float32),
        compiler_params=pltpu.CompilerParams(dimension_semantics=("parallel",)),
    )(xj, es, sh, w1, b1, w2, b2, Wtp)


def _pool(h, batch3, lw, lb):
    return pl.pallas_call(
        _pool_body,
        grid=(N // BN,),
        in_specs=[pl.BlockSpec((BN, DIM), lambda i: (i, 0)),
                  pl.BlockSpec((1, 1, BN), lambda i: (i, 0, 0)),
                  pl.BlockSpec((DIM, 1), lambda i: (0, 0)),
                  pl.BlockSpec((1, 1), lambda i: (0, 0))],
        out_specs=pl.BlockSpec((G, 1), lambda i: (0, 0)),
        out_shape=jax.ShapeDtypeStruct((G, 1), jnp.float32),
        scratch_shapes=[pltpu.VMEM((G, DIM + 1), jnp.float32)],
    )(h, batch3, lw, lb)


_MESH = plsc.VectorSubcoreMesh(core_axis_name="c", subcore_axis_name="s")


def _make_gather(e_part):
    nss_tot = e_part // SS

    @functools.partial(
        pl.kernel,
        out_type=jax.ShapeDtypeStruct((e_part, DIM), jnp.float32),
        mesh=_MESH,
        scratch_types=[pltpu.VMEM((SS,), jnp.int32),
                       pltpu.VMEM((SS, DIM), jnp.float32),
                       pltpu.SemaphoreType.DMA],
        compiler_params=pltpu.CompilerParams(use_tc_tiling_on_sc=False),
    )
    def gather_sc(h_hbm, src_hbm, out_hbm, idx_v, rows_v, sem):
        c = lax.axis_index("c")
        s = lax.axis_index("s")
        wid = s * NC + c
        nss = (nss_tot - wid + NW - 1) // NW

        def body(j, carry):
            base = (wid + j * NW) * SS
            pltpu.sync_copy(src_hbm.at[pl.ds(base, SS)], idx_v)
            cps = [pltpu.async_copy(h_hbm.at[idx_v.at[pl.ds(r * CHUNK, CHUNK)]],
                                    rows_v.at[pl.ds(r * CHUNK, CHUNK)], sem)
                   for r in range(SUPER)]
            for cp in cps:
                cp.wait()
            pltpu.sync_copy(rows_v, out_hbm.at[pl.ds(base, SS)])
            return carry

        lax.fori_loop(0, nss, body, 0)

    return gather_sc


def _make_scatter(e_parts):
    @functools.partial(
        pl.kernel,
        out_type=jax.ShapeDtypeStruct((N, DIM), jnp.float32),
        mesh=_MESH,
        scratch_types=[pltpu.VMEM((SS_W,), jnp.int32),
                       pltpu.VMEM((SUPER_W, CHUNK), jnp.int32),
                       pltpu.VMEM((SS_W, DIM), jnp.float32),
                       pltpu.VMEM((ROWCH, DIM), jnp.float32),
                       pltpu.VMEM_SHARED((HALF_PAD, DIM), jnp.float32),
                       pltpu.SemaphoreType.DMA,
                       pltpu.SemaphoreType.DMA],
        compiler_params=pltpu.CompilerParams(use_tc_tiling_on_sc=False),
    )
    def scatter_sc(h_hbm, msg0_hbm, msg1_hbm, dst_hbm, out_hbm, dst_v, idx_v,
                   msg_v, stage_v, acc_sh, sem, sem2):
        c = lax.axis_index("c")
        s = lax.axis_index("s")
        node_base = c * HALF

        nrow = (NROWCH - s + NS - 1) // NS

        def seed(j, carry):
            r = (s + j * NS) * ROWCH
            pltpu.sync_copy(h_hbm.at[pl.ds(node_base + r, ROWCH)], stage_v)
            pltpu.sync_copy(stage_v, acc_sh.at[pl.ds(r, ROWCH)])
            return carry

        lax.fori_loop(0, nrow, seed, 0)
        plsc.subcore_barrier()

        off = 0
        for p, e_part in enumerate(e_parts):
            msg_hbm = (msg0_hbm, msg1_hbm)[p]
            nss_w_tot = e_part // SS_W
            nss = (nss_w_tot - s + NS - 1) // NS
            d_off = off

            def body(j, carry, msg_hbm=msg_hbm, d_off=d_off):
                base = (s + j * NS) * SS_W
                cd = pltpu.async_copy(dst_hbm.at[pl.ds(d_off + base, SS_W)],
                                      dst_v, sem)
                cm = pltpu.async_copy(msg_hbm.at[pl.ds(base, SS_W)], msg_v, sem)
                cd.wait()
                for r in range(SUPER_W):
                    for q in range(CHUNK // 16):
                        v = dst_v[pl.ds(r * CHUNK + q * 16, 16)] - node_base
                        ok = (v >= 0) & (v < HALF)
                        idx_v[r, pl.ds(q * 16, 16)] = jnp.where(ok, v, HALF)
                cm.wait()
                cps = [pltpu.async_copy(msg_v.at[pl.ds(r * CHUNK, CHUNK)],
                                        acc_sh.at[idx_v.at[r]], sem2, add=True)
                       for r in range(SUPER_W)]
                for cp in cps:
                    cp.wait()
                return carry

            lax.fori_loop(0, nss, body, 0)
            off += e_part
        plsc.subcore_barrier()

        def drain(j, carry):
            r = (s + j * NS) * ROWCH
            pltpu.sync_copy(acc_sh.at[pl.ds(r, ROWCH)], stage_v)
            pltpu.sync_copy(stage_v, out_hbm.at[pl.ds(node_base + r, ROWCH)])
            return carry

        lax.fori_loop(0, nrow, drain, 0)

    return scatter_sc


_GATHER_SC = tuple(_make_gather(e) for e in EQ)
_SCATTER_SC = (_make_scatter(EQ[0:2]), _make_scatter(EQ[2:4]))


@jax.jit
def kernel(x, edge_index, edge_attr, batch, W_embed, b_embed, tp_w,
           mlp_w1, mlp_b1, mlp_w2, mlp_b2, lin_w, lin_b):
    f32 = jnp.float32
    t = jnp.arange(DIM)
    orig_of_T = (t % MUL) * 3 + t // MUL
    k = t // MUL
    u = t % MUL

    W_embT = W_embed[:, orig_of_T]
    b_embT = b_embed[orig_of_T].reshape(1, DIM)
    w1p = jnp.concatenate([mlp_w1, jnp.zeros((LAYERS, 1, 16), f32)], axis=1)
    b1r = mlp_b1.reshape(LAYERS, 1, 16)
    w2T = mlp_w2[:, :, orig_of_T]
    b2T = mlp_b2[:, orig_of_T].reshape(LAYERS, 1, DIM)
    tp_norm = 1.0 / jnp.sqrt(2.0 * MUL)
    I3 = jnp.eye(3, dtype=f32)
    Wtp = jnp.einsum('ij,luv->liujv', I3, tp_w).reshape(LAYERS, DIM, DIM) * tp_norm
    lwT = lin_w[orig_of_T]
    lbr = lin_b.reshape(1, 1)

    src = edge_index[0]
    dst = edge_index[1]
    batch3 = batch.reshape(N // BN, 1, BN)

    src_q = tuple(src[o:o + e] for o, e in zip(EQ_OFF, EQ))
    dst_h = (dst[:E0], dst[E0:])

    h = _embed(x, W_embT, b_embT)
    es, sh = _edgeprep(edge_attr)
    es_q = tuple(es[o:o + e] for o, e in zip(EQ_OFF, EQ))
    sh_q = tuple(sh[o:o + e] for o, e in zip(EQ_OFF, EQ))
    for l in range(LAYERS):
        xj = [_GATHER_SC[q](h, src_q[q]) for q in range(4)]
        m = [_msg(xj[q], es_q[q], sh_q[q],
                  w1p[l], b1r[l], w2T[l], b2T[l], Wtp[l]) for q in range(4)]
        h = _SCATTER_SC[0](h, m[0], m[1], dst_h[0])
        h = _SCATTER_SC[1](h, m[2], m[3], dst_h[1])
    out = _pool(h, batch3, lwT, lbr)
    return out.reshape(-1)


# speedup vs baseline: 2.2439x; 1.0467x over previous
import functools

import jax
import jax.numpy as jnp
from jax import lax
from jax.experimental import pallas as pl
from jax.experimental.pallas import tpu as pltpu
from jax.experimental.pallas import tpu_sc as plsc

N = 50000
E = 800000
G = 64
MUL = 16
DIM = 48
LAYERS = 4

NC = 2
NS = 16
NW = NC * NS

CHUNK = 128
SUPER = 10
SS = SUPER * CHUNK
NSS = E // SS
SUPER_W = 5
SS_W = SUPER_W * CHUNK
NSS_W = E // SS_W
HALF = N // NC
HALF_PAD = HALF + 8
ROWCH = 250
NROWCH = HALF // ROWCH

BE = 6400
BN = 1000

E0 = 409600
E1 = E - E0
EQ = (204800, 204800, 204800, 185600)
EQ_OFF = (0, 204800, 409600, 614400)


def _dot(a, b):
    return jnp.dot(a, b, preferred_element_type=jnp.float32,
                   precision=lax.Precision.HIGHEST)


def _embed_body(x_ref, w_ref, b_ref, o_ref):
    o_ref[...] = _dot(x_ref[...], w_ref[...]) + b_ref[...]


def _edgeprep_body(ea_ref, es_ref, sh_ref):
    ea = ea_ref[...]
    r = ea[:, 2:5]
    r2 = jnp.sum(r * r, axis=1, keepdims=True)
    z = jnp.zeros((BE, 1), jnp.float32)
    es_ref[...] = jnp.concatenate([ea[:, 0:2], r2, z], axis=1)
    inv = jnp.sqrt(3.0) / (jnp.sqrt(r2) + 1e-12)
    sh_ref[...] = jnp.concatenate([r * inv, z], axis=1)


def _dot_h(a, b):
    return jnp.dot(a, b, preferred_element_type=jnp.float32)


def _msg_body(xj_ref, es_ref, sh_ref, w1_ref, b1_ref, w2_ref, b2_ref,
              Wtp_ref, o_ref):
    es = es_ref[...]
    t1 = _dot_h(es, w1_ref[...]) + b1_ref[...]
    g1 = t1 * jax.nn.sigmoid(t1)
    gate = _dot_h(g1, w2_ref[...]) + b2_ref[...]
    xj = xj_ref[...]
    sh = sh_ref[...]
    a0, a1, a2 = xj[:, 0:16], xj[:, 16:32], xj[:, 32:48]
    b0, b1, b2 = sh[:, 0:1], sh[:, 1:2], sh[:, 2:3]
    cg = jnp.concatenate([a1 * b2 - a2 * b1,
                          a2 * b0 - a0 * b2,
                          a0 * b1 - a1 * b0], axis=1)
    tp = _dot_h(cg, Wtp_ref[...])
    o_ref[...] = gate * tp


def _pool_body(h_ref, b_ref, lw_ref, lb_ref, o_ref, acc_ref):
    i = pl.program_id(0)

    @pl.when(i == 0)
    def _():
        acc_ref[...] = jnp.zeros_like(acc_ref)

    bb = b_ref[0]
    gi = lax.broadcasted_iota(jnp.int32, (G, BN), 0)
    oh = (gi == bb).astype(jnp.float32)
    h_aug = jnp.concatenate(
        [h_ref[...], jnp.ones((BN, 1), jnp.float32)], axis=1)
    acc_ref[...] += _dot(oh, h_aug)

    @pl.when(i == pl.num_programs(0) - 1)
    def _():
        acc = acc_ref[...]
        pooled = acc[:, :DIM] / jnp.maximum(acc[:, DIM:DIM + 1], 1.0)
        o_ref[...] = _dot(pooled, lw_ref[...]) + lb_ref[...]


def _embed(x, W, b):
    return pl.pallas_call(
        _embed_body,
        grid=(N // BN,),
        in_specs=[pl.BlockSpec((BN, 3), lambda i: (i, 0)),
                  pl.BlockSpec((3, DIM), lambda i: (0, 0)),
                  pl.BlockSpec((1, DIM), lambda i: (0, 0))],
        out_specs=pl.BlockSpec((BN, DIM), lambda i: (i, 0)),
        out_shape=jax.ShapeDtypeStruct((N, DIM), jnp.float32),
        compiler_params=pltpu.CompilerParams(dimension_semantics=("parallel",)),
    )(x, W, b)


def _edgeprep(ea):
    return pl.pallas_call(
        _edgeprep_body,
        grid=(E // BE,),
        in_specs=[pl.BlockSpec((BE, 5), lambda i: (i, 0))],
        out_specs=[pl.BlockSpec((BE, 4), lambda i: (i, 0)),
                   pl.BlockSpec((BE, 4), lambda i: (i, 0))],
        out_shape=[jax.ShapeDtypeStruct((E, 4), jnp.float32),
                   jax.ShapeDtypeStruct((E, 4), jnp.float32)],
        compiler_params=pltpu.CompilerParams(dimension_semantics=("parallel",)),
    )(ea)


def _msg(xj, es, sh, w1, b1, w2, b2, Wtp):
    small = lambda shp: pl.BlockSpec(shp, lambda i: (0, 0))
    e_part = xj.shape[0]
    return pl.pallas_call(
        _msg_body,
        grid=(e_part // BE,),
        in_specs=[pl.BlockSpec((BE, DIM), lambda i: (i, 0)),
                  pl.BlockSpec((BE, 4), lambda i: (i, 0)),
                  pl.BlockSpec((BE, 4), lambda i: (i, 0)),
                  small((4, 16)), small((1, 16)), small((16, DIM)),
                  small((1, DIM)), small((DIM, DIM))],
        out_specs=pl.BlockSpec((BE, DIM), lambda i: (i, 0)),
        out_shape=jax.ShapeDtypeStruct((e_part, DIM), jnp.float32),
        compiler_params=pltpu.CompilerParams(dimension_semantics=("parallel",)),
    )(xj, es, sh, w1, b1, w2, b2, Wtp)


def _pool(h, batch3, lw, lb):
    return pl.pallas_call(
        _pool_body,
        grid=(N // BN,),
        in_specs=[pl.BlockSpec((BN, DIM), lambda i: (i, 0)),
                  pl.BlockSpec((1, 1, BN), lambda i: (i, 0, 0)),
                  pl.BlockSpec((DIM, 1), lambda i: (0, 0)),
                  pl.BlockSpec((1, 1), lambda i: (0, 0))],
        out_specs=pl.BlockSpec((G, 1), lambda i: (0, 0)),
        out_shape=jax.ShapeDtypeStruct((G, 1), jnp.float32),
        scratch_shapes=[pltpu.VMEM((G, DIM + 1), jnp.float32)],
    )(h, batch3, lw, lb)


_MESH = plsc.VectorSubcoreMesh(core_axis_name="c", subcore_axis_name="s")


def _make_gather(e_part):
    nss_tot = e_part // SS

    @functools.partial(
        pl.kernel,
        out_type=jax.ShapeDtypeStruct((e_part, DIM), jnp.float32),
        mesh=_MESH,
        scratch_types=[pltpu.VMEM((SS,), jnp.int32),
                       pltpu.VMEM((SS, DIM), jnp.float32),
                       pltpu.SemaphoreType.DMA],
        compiler_params=pltpu.CompilerParams(use_tc_tiling_on_sc=False),
    )
    def gather_sc(h_hbm, src_hbm, out_hbm, idx_v, rows_v, sem):
        c = lax.axis_index("c")
        s = lax.axis_index("s")
        wid = s * NC + c
        nss = (nss_tot - wid + NW - 1) // NW

        def body(j, carry):
            base = (wid + j * NW) * SS
            pltpu.sync_copy(src_hbm.at[pl.ds(base, SS)], idx_v)
            cps = [pltpu.async_copy(h_hbm.at[idx_v.at[pl.ds(r * CHUNK, CHUNK)]],
                                    rows_v.at[pl.ds(r * CHUNK, CHUNK)], sem)
                   for r in range(SUPER)]
            for cp in cps:
                cp.wait()
            pltpu.sync_copy(rows_v, out_hbm.at[pl.ds(base, SS)])
            return carry

        lax.fori_loop(0, nss, body, 0)

    return gather_sc


def _make_scatter(e_parts):
    @functools.partial(
        pl.kernel,
        out_type=jax.ShapeDtypeStruct((N, DIM), jnp.float32),
        mesh=_MESH,
        scratch_types=[pltpu.VMEM((SS_W,), jnp.int32),
                       pltpu.VMEM((SUPER_W, CHUNK), jnp.int32),
                       pltpu.VMEM((SS_W, DIM), jnp.float32),
                       pltpu.VMEM((ROWCH, DIM), jnp.float32),
                       pltpu.VMEM_SHARED((HALF_PAD, DIM), jnp.float32),
                       pltpu.SemaphoreType.DMA,
                       pltpu.SemaphoreType.DMA],
        compiler_params=pltpu.CompilerParams(use_tc_tiling_on_sc=False),
    )
    def scatter_sc(h_hbm, msg0_hbm, msg1_hbm, dst_hbm, out_hbm, dst_v, idx_v,
                   msg_v, stage_v, acc_sh, sem, sem2):
        c = lax.axis_index("c")
        s = lax.axis_index("s")
        node_base = c * HALF

        nrow = (NROWCH - s + NS - 1) // NS

        def seed(j, carry):
            r = (s + j * NS) * ROWCH
            pltpu.sync_copy(h_hbm.at[pl.ds(node_base + r, ROWCH)], stage_v)
            pltpu.sync_copy(stage_v, acc_sh.at[pl.ds(r, ROWCH)])
            return carry

        lax.fori_loop(0, nrow, seed, 0)
        plsc.subcore_barrier()

        off = 0
        for p, e_part in enumerate(e_parts):
            msg_hbm = (msg0_hbm, msg1_hbm)[p]
            nss_w_tot = e_part // SS_W
            nss = (nss_w_tot - s + NS - 1) // NS
            d_off = off

            def body(j, carry, msg_hbm=msg_hbm, d_off=d_off):
                base = (s + j * NS) * SS_W
                cd = pltpu.async_copy(dst_hbm.at[pl.ds(d_off + base, SS_W)],
                                      dst_v, sem)
                cm = pltpu.async_copy(msg_hbm.at[pl.ds(base, SS_W)], msg_v, sem)
                cd.wait()
                for r in range(SUPER_W):
                    for q in range(CHUNK // 16):
                        v = dst_v[pl.ds(r * CHUNK + q * 16, 16)] - node_base
                        ok = (v >= 0) & (v < HALF)
                        idx_v[r, pl.ds(q * 16, 16)] = jnp.where(ok, v, HALF)
                cm.wait()
                cps = [pltpu.async_copy(msg_v.at[pl.ds(r * CHUNK, CHUNK)],
                                        acc_sh.at[idx_v.at[r]], sem2, add=True)
                       for r in range(SUPER_W)]
                for cp in cps:
                    cp.wait()
                return carry

            lax.fori_loop(0, nss, body, 0)
            off += e_part
        plsc.subcore_barrier()

        def drain(j, carry):
            r = (s + j * NS) * ROWCH
            pltpu.sync_copy(acc_sh.at[pl.ds(r, ROWCH)], stage_v)
            pltpu.sync_copy(stage_v, out_hbm.at[pl.ds(node_base + r, ROWCH)])
            return carry

        lax.fori_loop(0, nrow, drain, 0)

    return scatter_sc


_GATHER_SC = tuple(_make_gather(e) for e in EQ)
_SCATTER_SC = (_make_scatter(EQ[0:2]), _make_scatter(EQ[2:4]))


@jax.jit
def kernel(x, edge_index, edge_attr, batch, W_embed, b_embed, tp_w,
           mlp_w1, mlp_b1, mlp_w2, mlp_b2, lin_w, lin_b):
    f32 = jnp.float32
    t = jnp.arange(DIM)
    orig_of_T = (t % MUL) * 3 + t // MUL
    k = t // MUL
    u = t % MUL

    W_embT = W_embed[:, orig_of_T]
    b_embT = b_embed[orig_of_T].reshape(1, DIM)
    w1p = jnp.concatenate([mlp_w1, jnp.zeros((LAYERS, 1, 16), f32)], axis=1)
    b1r = mlp_b1.reshape(LAYERS, 1, 16)
    w2T = mlp_w2[:, :, orig_of_T]
    b2T = mlp_b2[:, orig_of_T].reshape(LAYERS, 1, DIM)
    tp_norm = 1.0 / jnp.sqrt(2.0 * MUL)
    I3 = jnp.eye(3, dtype=f32)
    Wtp = jnp.einsum('ij,luv->liujv', I3, tp_w).reshape(LAYERS, DIM, DIM) * tp_norm
    lwT = lin_w[orig_of_T]
    lbr = lin_b.reshape(1, 1)

    src = edge_index[0]
    dst = edge_index[1]
    batch3 = batch.reshape(N // BN, 1, BN)

    src_q = tuple(src[o:o + e] for o, e in zip(EQ_OFF, EQ))
    dst_h = (dst[:E0], dst[E0:])

    h = _embed(x, W_embT, b_embT)
    es, sh = _edgeprep(edge_attr)
    es_q = tuple(es[o:o + e] for o, e in zip(EQ_OFF, EQ))
    sh_q = tuple(sh[o:o + e] for o, e in zip(EQ_OFF, EQ))
    for l in range(LAYERS):
        xj = [_GATHER_SC[q](h, src_q[q]) for q in range(4)]
        m = [_msg(xj[q], es_q[q], sh_q[q],
                  w1p[l], b1r[l], w2T[l], b2T[l], Wtp[l]) for q in range(4)]
        h = _SCATTER_SC[0](h, m[0], m[1], dst_h[0])
        h = _SCATTER_SC[1](h, m[2], m[3], dst_h[1])
    out = _pool(h, batch3, lwT, lbr)
    return out.reshape(-1)
